# baseline (speedup 1.0000x reference)
import functools

import jax
import jax.numpy as jnp
from jax import lax
from jax.experimental import pallas as pl
from jax.experimental.pallas import tpu as pltpu
from jax.experimental.pallas import tpu_sc as plsc

I_DIM = 1000000
O_DIM = 32
WORD_DIM = 64
B, L = 4096, 200
N_TOK = B * L
NW = 32
HALF_TOK = N_TOK // 2
H_PER_W = HALF_TOK // NW
CHUNK = 512
N_CHUNK = H_PER_W // CHUNK
C_ROWS = N_TOK // 4
CH_ROWS = C_ROWS // 2

X_BLK = 32768
QW = X_BLK // 4
N_XBLK = -(-I_DIM // X_BLK)
T_LINES = N_XBLK * QW
T_ROWS = T_LINES * 4


def _table_relayout_tc(table_t):
    def body(t_ref, o_ref):
        eye128 = jnp.eye(128, dtype=jnp.float32)
        tb = t_ref[...]
        acc = None
        for q in range(4):
            part = jax.lax.dot_general(
                tb[:, q * QW:(q + 1) * QW],
                eye128[q * O_DIM:(q + 1) * O_DIM],
                (((0,), (0,)), ((), ())),
                preferred_element_type=jnp.float32,
            )
            acc = part if acc is None else acc + part
        o_ref[...] = acc

    return pl.pallas_call(
        body,
        grid=(N_XBLK,),
        in_specs=[pl.BlockSpec((O_DIM, X_BLK), lambda i: (0, i))],
        out_specs=pl.BlockSpec((QW, 128), lambda i: (i, 0)),
        out_shape=jax.ShapeDtypeStruct((T_LINES, 128), jnp.float32),
        compiler_params=pltpu.CompilerParams(
            fuse_transposed_lhs_in_matmul=True,
            vmem_limit_bytes=50 * 1024 * 1024,
        ),
    )(table_t)


def _gather_sc(table_lin, idx_mapped, half):
    mesh = plsc.VectorSubcoreMesh(core_axis_name="c", subcore_axis_name="s")
    half_base = half * HALF_TOK
    row_base = half * CH_ROWS

    @functools.partial(
        pl.kernel,
        mesh=mesh,
        out_type=jax.ShapeDtypeStruct((CH_ROWS, 128), jnp.float32),
        scratch_types=[
            pltpu.VMEM((CHUNK,), jnp.int32),
            pltpu.VMEM((CHUNK, O_DIM), jnp.float32),
            pltpu.SemaphoreType.DMA,
        ],
        compiler_params=pltpu.CompilerParams(use_tc_tiling_on_sc=False),
    )
    def gather_kernel(table_hbm, idx_hbm, out_hbm, idx_v, rows_v, sem):
        wid = lax.axis_index("s") * 2 + lax.axis_index("c")
        base = half_base + wid * H_PER_W

        def body(k, carry):
            u0 = base + k * CHUNK
            row0 = (u0 // B) * 1024 + (u0 % 1024) - row_base
            lane0 = ((u0 % B) // 1024) * O_DIM
            pltpu.sync_copy(idx_hbm.at[pl.ds(u0, CHUNK)], idx_v)
            pltpu.async_copy(table_hbm.at[idx_v], rows_v, sem).wait()
            pltpu.sync_copy(
                rows_v,
                out_hbm.at[pl.ds(row0, CHUNK), pl.ds(lane0, O_DIM)],
            )
            return carry

        lax.fori_loop(0, N_CHUNK, body, 0)

    return gather_kernel(table_lin, idx_mapped)


L_BLK = 5
LH = L // 2
HGRID = LH // L_BLK


def _proj_add_tc(prev_out, x2t, ch, word_W, half):
    off = half * HGRID

    def body(*refs):
        x2_ref, c_ref, w_ref, o_ref = refs[-4:]
        eye = jnp.eye(O_DIM, dtype=jnp.float32)
        for s in range(L_BLK):
            xb = x2_ref[s]
            w = jax.lax.dot_general(
                w_ref[...], xb, (((0,), (0,)), ((), ())),
                preferred_element_type=jnp.float32,
            )
            cb = c_ref[pl.ds(s * 1024, 1024), :]
            for j in range(4):
                ct_j = jax.lax.dot_general(
                    eye, cb[:, j * O_DIM:(j + 1) * O_DIM],
                    (((1,), (1,)), ((), ())),
                    preferred_element_type=jnp.float32,
                )
                o_ref[s, :, pl.ds(j * 1024, 1024)] = (
                    w[:, j * 1024:(j + 1) * 1024] + ct_j
                )

    specs = [
        pl.BlockSpec((L_BLK, WORD_DIM, B), lambda i, off=off: (i + off, 0, 0)),
        pl.BlockSpec((L_BLK * 1024, 128), lambda i: (i, 0)),
        pl.BlockSpec((WORD_DIM, O_DIM), lambda i: (0, 0)),
    ]
    args = (x2t, ch, word_W)
    aliases = {}
    if prev_out is not None:
        specs = [pl.BlockSpec(memory_space=pl.ANY)] + specs
        args = (prev_out,) + args
        aliases = {0: 0}
    return pl.pallas_call(
        body,
        grid=(HGRID,),
        in_specs=specs,
        out_specs=pl.BlockSpec((L_BLK, O_DIM, B), lambda i, off=off: (i + off, 0, 0)),
        out_shape=jax.ShapeDtypeStruct((L, O_DIM, B), jnp.float32),
        input_output_aliases=aliases,
        compiler_params=pltpu.CompilerParams(fuse_transposed_lhs_in_matmul=True),
    )(*args)


def kernel(x1, x2, char_table, word_W):
    idx_lb = jnp.transpose(x1, (1, 0)).reshape(N_TOK)
    x2t = jnp.transpose(x2, (1, 2, 0))
    table_t = jnp.transpose(char_table, (1, 0))

    t128 = _table_relayout_tc(table_t)
    table_lin = t128.reshape(T_ROWS, O_DIM)

    xsh = X_BLK.bit_length() - 1
    qsh = QW.bit_length() - 1
    k = idx_lb >> xsh
    p = idx_lb & (X_BLK - 1)
    q = p >> qsh
    r = p & (QW - 1)
    idx_mapped = ((k << qsh) + r) * 4 + q

    cA = _gather_sc(table_lin, idx_mapped, 0)
    cB = _gather_sc(table_lin, idx_mapped, 1)
    outA = _proj_add_tc(None, x2t, cA, word_W, 0)
    out_t = _proj_add_tc(outA, x2t, cB, word_W, 1)
    return jnp.transpose(out_t, (2, 0, 1))


# speedup vs baseline: 1.0482x; 1.0482x over previous
import functools

import jax
import jax.numpy as jnp
from jax import lax
from jax.experimental import pallas as pl
from jax.experimental.pallas import tpu as pltpu
from jax.experimental.pallas import tpu_sc as plsc

I_DIM = 1000000
O_DIM = 32
WORD_DIM = 64
B, L = 4096, 200
N_TOK = B * L
NW = 32
HALF_TOK = N_TOK // 2
H_PER_W = HALF_TOK // NW
CHUNK = 512
N_CHUNK = H_PER_W // CHUNK
C_ROWS = N_TOK // 4
CH_ROWS = C_ROWS // 2

X_BLK = 32768
QW = X_BLK // 4
N_XBLK = -(-I_DIM // X_BLK)
T_LINES = N_XBLK * QW
T_ROWS = T_LINES * 4


def _table_relayout_tc(table_t):
    def body(t_ref, o_ref):
        eye128 = jnp.eye(128, dtype=jnp.float32)
        tb = t_ref[...]
        acc = None
        for q in range(4):
            part = jax.lax.dot_general(
                tb[:, q * QW:(q + 1) * QW],
                eye128[q * O_DIM:(q + 1) * O_DIM],
                (((0,), (0,)), ((), ())),
                preferred_element_type=jnp.float32,
            )
            acc = part if acc is None else acc + part
        o_ref[...] = acc

    return pl.pallas_call(
        body,
        grid=(N_XBLK,),
        in_specs=[pl.BlockSpec((O_DIM, X_BLK), lambda i: (0, i))],
        out_specs=pl.BlockSpec((QW, 128), lambda i: (i, 0)),
        out_shape=jax.ShapeDtypeStruct((T_LINES, 128), jnp.float32),
        compiler_params=pltpu.CompilerParams(
            fuse_transposed_lhs_in_matmul=True,
            vmem_limit_bytes=50 * 1024 * 1024,
        ),
    )(table_t)


def _gather_sc(table_lin, idx_mapped, half):
    mesh = plsc.VectorSubcoreMesh(core_axis_name="c", subcore_axis_name="s")
    half_base = half * HALF_TOK
    row_base = half * CH_ROWS

    @functools.partial(
        pl.kernel,
        mesh=mesh,
        out_type=jax.ShapeDtypeStruct((CH_ROWS, 128), jnp.float32),
        scratch_types=[
            pltpu.VMEM((H_PER_W,), jnp.int32),
            pltpu.VMEM((CHUNK, O_DIM), jnp.float32),
            pltpu.SemaphoreType.DMA,
        ],
        compiler_params=pltpu.CompilerParams(use_tc_tiling_on_sc=False),
    )
    def gather_kernel(table_hbm, idx_hbm, out_hbm, idx_v, rows_v, sem):
        wid = lax.axis_index("s") * 2 + lax.axis_index("c")
        base = half_base + wid * H_PER_W
        pltpu.sync_copy(idx_hbm.at[pl.ds(base, H_PER_W)], idx_v)

        def body(k, carry):
            u0 = base + k * CHUNK
            row0 = (u0 // B) * 1024 + (u0 % 1024) - row_base
            lane0 = ((u0 % B) // 1024) * O_DIM
            pltpu.async_copy(
                table_hbm.at[idx_v.at[pl.ds(k * CHUNK, CHUNK)]], rows_v, sem
            ).wait()
            pltpu.sync_copy(
                rows_v,
                out_hbm.at[pl.ds(row0, CHUNK), pl.ds(lane0, O_DIM)],
            )
            return carry

        lax.fori_loop(0, N_CHUNK, body, 0)

    return gather_kernel(table_lin, idx_mapped)


L_BLK = 5
LH = L // 2
HGRID = LH // L_BLK


def _proj_add_tc(prev_out, x2t, ch, word_W, half):
    off = half * HGRID

    def body(*refs):
        x2_ref, c_ref, w_ref, o_ref = refs[-4:]
        eye = jnp.eye(O_DIM, dtype=jnp.float32)
        for s in range(L_BLK):
            xb = x2_ref[s]
            w = jax.lax.dot_general(
                w_ref[...], xb, (((0,), (0,)), ((), ())),
                preferred_element_type=jnp.float32,
            )
            cb = c_ref[pl.ds(s * 1024, 1024), :]
            for j in range(4):
                ct_j = jax.lax.dot_general(
                    eye, cb[:, j * O_DIM:(j + 1) * O_DIM],
                    (((1,), (1,)), ((), ())),
                    preferred_element_type=jnp.float32,
                )
                o_ref[s, :, pl.ds(j * 1024, 1024)] = (
                    w[:, j * 1024:(j + 1) * 1024] + ct_j
                )

    specs = [
        pl.BlockSpec((L_BLK, WORD_DIM, B), lambda i, off=off: (i + off, 0, 0)),
        pl.BlockSpec((L_BLK * 1024, 128), lambda i: (i, 0)),
        pl.BlockSpec((WORD_DIM, O_DIM), lambda i: (0, 0)),
    ]
    args = (x2t, ch, word_W)
    aliases = {}
    if prev_out is not None:
        specs = [pl.BlockSpec(memory_space=pl.ANY)] + specs
        args = (prev_out,) + args
        aliases = {0: 0}
    return pl.pallas_call(
        body,
        grid=(HGRID,),
        in_specs=specs,
        out_specs=pl.BlockSpec((L_BLK, O_DIM, B), lambda i, off=off: (i + off, 0, 0)),
        out_shape=jax.ShapeDtypeStruct((L, O_DIM, B), jnp.float32),
        input_output_aliases=aliases,
        compiler_params=pltpu.CompilerParams(fuse_transposed_lhs_in_matmul=True),
    )(*args)


def kernel(x1, x2, char_table, word_W):
    idx_lb = jnp.transpose(x1, (1, 0)).reshape(N_TOK)
    x2t = jnp.transpose(x2, (1, 2, 0))
    table_t = jnp.transpose(char_table, (1, 0))

    t128 = _table_relayout_tc(table_t)
    table_lin = t128.reshape(T_ROWS, O_DIM)

    xsh = X_BLK.bit_length() - 1
    qsh = QW.bit_length() - 1
    k = idx_lb >> xsh
    p = idx_lb & (X_BLK - 1)
    q = p >> qsh
    r = p & (QW - 1)
    idx_mapped = ((k << qsh) + r) * 4 + q

    cA = _gather_sc(table_lin, idx_mapped, 0)
    cB = _gather_sc(table_lin, idx_mapped, 1)
    outA = _proj_add_tc(None, x2t, cA, word_W, 0)
    out_t = _proj_add_tc(outA, x2t, cB, word_W, 1)
    return jnp.transpose(out_t, (2, 0, 1))


# speedup vs baseline: 1.0574x; 1.0088x over previous
import functools

import jax
import jax.numpy as jnp
from jax import lax
from jax.experimental import pallas as pl
from jax.experimental.pallas import tpu as pltpu
from jax.experimental.pallas import tpu_sc as plsc

I_DIM = 1000000
O_DIM = 32
WORD_DIM = 64
B, L = 4096, 200
N_TOK = B * L
NW = 32
HALF_TOK = N_TOK // 2
H_PER_W = HALF_TOK // NW
CHUNK = 512
N_CHUNK = H_PER_W // CHUNK
C_ROWS = N_TOK // 4
CH_ROWS = C_ROWS // 2

X_BLK = 32768
QW = X_BLK // 4
N_XBLK = -(-I_DIM // X_BLK)
T_LINES = N_XBLK * QW
T_ROWS = T_LINES * 4


def _table_relayout_tc(table_t):
    def body(t_ref, o_ref):
        eye128 = jnp.eye(128, dtype=jnp.float32)
        tb = t_ref[...]
        acc = None
        for q in range(4):
            part = jax.lax.dot_general(
                tb[:, q * QW:(q + 1) * QW],
                eye128[q * O_DIM:(q + 1) * O_DIM],
                (((0,), (0,)), ((), ())),
                preferred_element_type=jnp.float32,
            )
            acc = part if acc is None else acc + part
        o_ref[...] = acc

    return pl.pallas_call(
        body,
        grid=(N_XBLK,),
        in_specs=[pl.BlockSpec((O_DIM, X_BLK), lambda i: (0, i))],
        out_specs=pl.BlockSpec((QW, 128), lambda i: (i, 0)),
        out_shape=jax.ShapeDtypeStruct((T_LINES, 128), jnp.float32),
        compiler_params=pltpu.CompilerParams(
            fuse_transposed_lhs_in_matmul=True,
            vmem_limit_bytes=50 * 1024 * 1024,
        ),
    )(table_t)


def _gather_sc(table_lin, idx_mapped, half):
    mesh = plsc.VectorSubcoreMesh(core_axis_name="c", subcore_axis_name="s")
    half_base = half * HALF_TOK
    row_base = half * CH_ROWS

    @functools.partial(
        pl.kernel,
        mesh=mesh,
        out_type=jax.ShapeDtypeStruct((CH_ROWS, 128), jnp.float32),
        scratch_types=[
            pltpu.VMEM((H_PER_W,), jnp.int32),
            pltpu.VMEM((CHUNK, O_DIM), jnp.float32),
            pltpu.SemaphoreType.DMA,
        ],
        compiler_params=pltpu.CompilerParams(use_tc_tiling_on_sc=False),
    )
    def gather_kernel(table_hbm, idx_hbm, out_hbm, idx_v, rows_v, sem):
        wid = lax.axis_index("s") * 2 + lax.axis_index("c")
        base = half_base + wid * H_PER_W
        pltpu.sync_copy(idx_hbm.at[pl.ds(base, H_PER_W)], idx_v)

        def body(k, carry):
            u0 = base + k * CHUNK
            row0 = (u0 // B) * 1024 + (u0 % 1024) - row_base
            lane0 = ((u0 % B) // 1024) * O_DIM
            pltpu.async_copy(
                table_hbm.at[idx_v.at[pl.ds(k * CHUNK, CHUNK)]], rows_v, sem
            ).wait()
            pltpu.sync_copy(
                rows_v,
                out_hbm.at[pl.ds(row0, CHUNK), pl.ds(lane0, O_DIM)],
            )
            return carry

        lax.fori_loop(0, N_CHUNK, body, 0)

    return gather_kernel(table_lin, idx_mapped)


L_BLK = 10
LH = L // 2
HGRID = LH // L_BLK


def _proj_add_tc(prev_out, x2t, ch, word_W, half):
    off = half * HGRID

    def body(*refs):
        x2_ref, c_ref, w_ref, o_ref = refs[-4:]
        eye = jnp.eye(O_DIM, dtype=jnp.float32)
        for s in range(L_BLK):
            xb = x2_ref[s]
            w = jax.lax.dot_general(
                w_ref[...], xb, (((0,), (0,)), ((), ())),
                preferred_element_type=jnp.float32,
            )
            cb = c_ref[pl.ds(s * 1024, 1024), :]
            for j in range(4):
                ct_j = jax.lax.dot_general(
                    eye, cb[:, j * O_DIM:(j + 1) * O_DIM],
                    (((1,), (1,)), ((), ())),
                    preferred_element_type=jnp.float32,
                )
                o_ref[s, :, pl.ds(j * 1024, 1024)] = (
                    w[:, j * 1024:(j + 1) * 1024] + ct_j
                )

    specs = [
        pl.BlockSpec((L_BLK, WORD_DIM, B), lambda i, off=off: (i + off, 0, 0)),
        pl.BlockSpec((L_BLK * 1024, 128), lambda i: (i, 0)),
        pl.BlockSpec((WORD_DIM, O_DIM), lambda i: (0, 0)),
    ]
    args = (x2t, ch, word_W)
    aliases = {}
    if prev_out is not None:
        specs = [pl.BlockSpec(memory_space=pl.ANY)] + specs
        args = (prev_out,) + args
        aliases = {0: 0}
    return pl.pallas_call(
        body,
        grid=(HGRID,),
        in_specs=specs,
        out_specs=pl.BlockSpec((L_BLK, O_DIM, B), lambda i, off=off: (i + off, 0, 0)),
        out_shape=jax.ShapeDtypeStruct((L, O_DIM, B), jnp.float32),
        input_output_aliases=aliases,
        compiler_params=pltpu.CompilerParams(
            fuse_transposed_lhs_in_matmul=True,
            vmem_limit_bytes=50 * 1024 * 1024,
        ),
    )(*args)


def kernel(x1, x2, char_table, word_W):
    idx_lb = jnp.transpose(x1, (1, 0)).reshape(N_TOK)
    x2t = jnp.transpose(x2, (1, 2, 0))
    table_t = jnp.transpose(char_table, (1, 0))

    t128 = _table_relayout_tc(table_t)
    table_lin = t128.reshape(T_ROWS, O_DIM)

    xsh = X_BLK.bit_length() - 1
    qsh = QW.bit_length() - 1
    k = idx_lb >> xsh
    p = idx_lb & (X_BLK - 1)
    q = p >> qsh
    r = p & (QW - 1)
    idx_mapped = ((k << qsh) + r) * 4 + q

    cA = _gather_sc(table_lin, idx_mapped, 0)
    cB = _gather_sc(table_lin, idx_mapped, 1)
    outA = _proj_add_tc(None, x2t, cA, word_W, 0)
    out_t = _proj_add_tc(outA, x2t, cB, word_W, 1)
    return jnp.transpose(out_t, (2, 0, 1))
